# Initial kernel scaffold; baseline (speedup 1.0000x reference)
import jax
import jax.numpy as jnp
from jax.experimental import pallas as pl


def kernel(inp_word, inp_char, inp_pos, inp_mask, word_emb, char_emb, pos_emb, W_ctx, b_ctx, W_dec, b_dec, psr_weight):
    raise NotImplementedError("write your pallas kernel here")


# speedup vs baseline: 9.7084x; 9.7084x over previous
import numpy as np
import jax
import jax.numpy as jnp
from jax.experimental import pallas as pl
from jax.experimental.pallas import tpu as pltpu
from jax.experimental.pallas import tpu_sc as plsc

_B, _L = 4, 2048
_N = _B * _L
_V, _DW, _DC, _DP = 32768, 256, 64, 64
_CV, _CL = 100, 16
_E, _M = 16, 1024
_HS = 1024
_DPSR = 256
_UNK = 0
_ENT_PENALTY = 0.01

_TB = 256
_NBLK = _N // _TB
_G = _NBLK + _E - 1
_TBA = 512

_TINY = np.float32(np.finfo(np.float32).tiny)
_ENT_SCALE = np.float32(-_ENT_PENALTY / (_N * _M))


_ROTS = ((13, 15, 26, 6), (17, 29, 16, 24))
_KS_SCHED = ((1, 2, 1), (2, 0, 2), (0, 1, 3), (1, 2, 4), (2, 0, 5))


def _np_threefry2x32(k1, k2, x0, x1):
    ks = (np.uint32(k1), np.uint32(k2),
          np.uint32(k1) ^ np.uint32(k2) ^ np.uint32(0x1BD11BDA))
    x0 = np.uint32(np.uint64(x0) + ks[0])
    x1 = np.uint32(np.uint64(x1) + ks[1])
    for i, (a, b, c) in enumerate(_KS_SCHED):
        for r in _ROTS[i % 2]:
            x0 = np.uint32((np.uint64(x0) + np.uint64(x1)) & 0xFFFFFFFF)
            x1 = np.uint32(((x1 << np.uint32(r)) | (x1 >> np.uint32(32 - r))))
            x1 = x0 ^ x1
        x0 = np.uint32((np.uint64(x0) + np.uint64(ks[a])) & 0xFFFFFFFF)
        x1 = np.uint32((np.uint64(x1) + np.uint64(ks[b]) + c) & 0xFFFFFFFF)
    return x0, x1


def _fold_key(e):
    return _np_threefry2x32(np.uint32(0), np.uint32(42), np.uint32(0), np.uint32(e))


_KEY_WORDS = np.array([_fold_key(e) for e in range(_E)], dtype=np.uint32)
_KEY1_I32 = _KEY_WORDS[:, 0].view(np.int32)
_KEY2_I32 = _KEY_WORDS[:, 1].view(np.int32)


def _routing(posf):
    n = posf.shape[0]
    oh = (posf[:, None] == jnp.arange(_E, dtype=posf.dtype)[None, :]).astype(jnp.int32)
    occ = jnp.cumsum(oh, axis=0)
    counts = occ[-1]
    starts = jnp.concatenate([jnp.zeros((1,), jnp.int32),
                              jnp.cumsum(counts)[:-1].astype(jnp.int32)])
    rank = starts[posf] + jnp.take_along_axis(occ, posf[:, None], axis=1)[:, 0] - 1
    perm = jnp.zeros((n,), jnp.int32).at[rank].set(jnp.arange(n, dtype=jnp.int32))

    ends = starts + counts
    fb = starts // _TB
    lb = jnp.where(counts > 0, (ends - 1) // _TB, fb - 1)
    nb = jnp.where(counts > 0, lb - fb + 1, 0)
    gs = jnp.concatenate([jnp.zeros((1,), jnp.int32),
                          jnp.cumsum(nb)[:-1].astype(jnp.int32)])
    total = gs[-1] + nb[-1]
    gid = jnp.arange(_G, dtype=jnp.int32)
    ge = gs + nb
    e_g = jnp.minimum(jnp.sum((gid[:, None] >= ge[None, :]).astype(jnp.int32), axis=1),
                      _E - 1)
    valid = (gid < total).astype(jnp.int32)
    b_g = fb[e_g] + (gid - gs[e_g])
    b_g = jnp.where(valid == 1, b_g, _NBLK - 1).astype(jnp.int32)
    prev_b = jnp.concatenate([jnp.full((1,), -1, jnp.int32), b_g[:-1]])
    first = ((b_g != prev_b) & (valid == 1)).astype(jnp.int32)
    k1 = jnp.asarray(_KEY1_I32)[e_g]
    k2 = jnp.asarray(_KEY2_I32)[e_g]
    return rank, perm, e_g.astype(jnp.int32), b_g, first, valid, k1, k2


def _sc_gather(table, idx, window):
    n = idx.shape[0]
    width = table.shape[1]
    idx2 = idx.reshape(1, n)
    mesh = plsc.VectorSubcoreMesh(core_axis_name="core", subcore_axis_name="subcore")

    @pl.kernel(out_type=jax.ShapeDtypeStruct((n, width), table.dtype), mesh=mesh)
    def gather_kernel(x_hbm, i_hbm, o_hbm):
        def body(i_vmem, o_vmem):
            pltpu.sync_copy(x_hbm.at[i_vmem.at[0]], o_vmem)

        pltpu.emit_pipeline(
            body,
            grid=(n // window,),
            in_specs=[pl.BlockSpec((1, window), lambda i: (0, i))],
            out_specs=[pl.BlockSpec((window, width), lambda i: (i, 0))],
            core_axis_name="subcore",
            dimension_semantics=(pltpu.PARALLEL,),
        )(i_hbm, o_hbm)

    return gather_kernel(table, idx2)


def _ctx_body(we_ref, ch_ref, pos_ref, mask_ref, wctx_ref, bctx_ref,
              cemb_ref, pemb_ref, out_ref):
    f32 = jnp.float32
    acc = jnp.dot(we_ref[...], wctx_ref[0:_DW, :], preferred_element_type=f32)
    iota = jax.lax.broadcasted_iota(jnp.int32, (_TBA, 128), 1)
    ids = ch_ref[...]
    cnt = jnp.zeros((_TBA, 128), f32)
    for l in range(_CL):
        cnt = cnt + (ids[:, l:l + 1] == iota).astype(f32)
    ce = jnp.dot(cnt, cemb_ref[...], preferred_element_type=f32) * f32(1.0 / _CL)
    acc = acc + jnp.dot(ce, wctx_ref[_DW:_DW + _DC, :], preferred_element_type=f32)
    poh = (pos_ref[...] == iota).astype(f32)
    pe = jnp.dot(poh, pemb_ref[...], preferred_element_type=f32)
    acc = acc + jnp.dot(pe, wctx_ref[_DW + _DC:_DW + _DC + _DP, :],
                        preferred_element_type=f32)
    out_ref[...] = jnp.tanh(acc * mask_ref[...] + bctx_ref[...])


def _ctx_call(we_sorted, ch_sorted, pos_col, mask_col, W_ctx, b_ctx,
              cemb_p, pemb_p, interpret=False):
    nsteps = _N // _TBA
    return pl.pallas_call(
        _ctx_body,
        grid=(nsteps,),
        in_specs=[
            pl.BlockSpec((_TBA, _DW), lambda i: (i, 0)),
            pl.BlockSpec((_TBA, _CL), lambda i: (i, 0)),
            pl.BlockSpec((_TBA, 1), lambda i: (i, 0)),
            pl.BlockSpec((_TBA, 1), lambda i: (i, 0)),
            pl.BlockSpec((_DW + _DC + _DP, _HS), lambda i: (0, 0)),
            pl.BlockSpec((1, _HS), lambda i: (0, 0)),
            pl.BlockSpec((128, _DC), lambda i: (0, 0)),
            pl.BlockSpec((128, _DP), lambda i: (0, 0)),
        ],
        out_specs=pl.BlockSpec((_TBA, _HS), lambda i: (i, 0)),
        out_shape=jax.ShapeDtypeStruct((_N, _HS), jnp.float32),
        compiler_params=pltpu.CompilerParams(
            dimension_semantics=("arbitrary",)),
        interpret=interpret,
    )(we_sorted, ch_sorted, pos_col, mask_col, W_ctx, b_ctx, cemb_p, pemb_p)


def _decode_body(eg, bg, fi, va, k1a, k2a,
                 ctx_ref, wdec_ref, bdec_ref, tab_ref, unk_ref,
                 pos_ref, word_ref, perm_ref,
                 emb_ref, wout_ref, ent_ref):
    f32 = jnp.float32
    g = pl.program_id(0)
    e = eg[g]
    first = fi[g]
    valid = va[g]

    logits = jnp.dot(ctx_ref[...], wdec_ref[0], preferred_element_type=f32)
    logits = logits + bdec_ref[0]

    rowmask = (pos_ref[...] == e) & (valid == 1)

    m = jnp.max(logits, axis=1, keepdims=True)
    ex = jnp.exp(logits - m)
    s = jnp.sum(ex, axis=1, keepdims=True)
    logp = logits - m - jnp.log(s)
    p = ex / s
    hrow = jnp.sum(-logp * p, axis=1, keepdims=True)
    hsum = jnp.sum(jnp.where(rowmask, hrow, f32(0.0)))

    @pl.when(g == 0)
    def _():
        ent_ref[...] = jnp.zeros((1, 1), f32)

    ent_ref[...] = ent_ref[...] + jnp.reshape(hsum * _ENT_SCALE, (1, 1))

    u32 = jnp.uint32
    i_orig = perm_ref[...]
    col = jax.lax.broadcasted_iota(jnp.int32, (_TB, _M), 1)
    idx = (i_orig * _M + col).astype(u32)
    k1 = k1a[g].astype(u32)
    k2 = k2a[g].astype(u32)
    ks = (k1, k2, k1 ^ k2 ^ u32(0x1BD11BDA))
    x0 = jnp.broadcast_to(k1, (_TB, _M))
    x1 = idx + k2
    for i, (a, b, c) in enumerate(_KS_SCHED):
        for r in _ROTS[i % 2]:
            x0 = x0 + x1
            x1 = (x1 << u32(r)) | (x1 >> u32(32 - r))
            x1 = x0 ^ x1
        x0 = x0 + ks[a]
        x1 = x1 + ks[b] + u32(c)
    bits = x0 ^ x1
    fbits = (bits >> u32(9)) | u32(0x3F800000)
    fl = jax.lax.bitcast_convert_type(fbits, f32) - f32(1.0)
    u = jnp.maximum(_TINY, fl + _TINY)
    gmb = -jnp.log(-jnp.log(u))

    z = logits + gmb
    zm = jnp.max(z, axis=1, keepdims=True)
    ez = jnp.exp(z - zm)
    sz = jnp.sum(ez, axis=1, keepdims=True)
    spt = ez / sz
    emb = jnp.dot(spt, tab_ref[...], preferred_element_type=f32)

    mx = jnp.max(spt, axis=1, keepdims=True)
    big = jnp.int32(_M)
    am = jnp.min(jnp.where(spt == mx, col, big), axis=1, keepdims=True)
    word = am + e * _M

    avoid = (word == word_ref[...]) & rowmask
    word = jnp.where(avoid, jnp.int32(_UNK), word)
    emb = jnp.where(avoid, unk_ref[...], emb)

    emb_c = jnp.where(rowmask, emb, f32(0.0))
    word_c = jnp.where(rowmask, jnp.broadcast_to(word, (_TB, 128)), jnp.int32(0))

    @pl.when(first == 1)
    def _():
        emb_ref[...] = emb_c
        wout_ref[...] = word_c

    @pl.when(first == 0)
    def _():
        emb_ref[...] = emb_ref[...] + emb_c
        wout_ref[...] = wout_ref[...] + word_c


def _decode_call(e_g, b_g, first, valid, k1, k2, ctx_sorted, W_dec, b_dec,
                 psr_weight, unk_row, pos_col, word_col, perm_col,
                 interpret=False):
    grid_spec = pltpu.PrefetchScalarGridSpec(
        num_scalar_prefetch=6,
        grid=(_G,),
        in_specs=[
            pl.BlockSpec((_TB, _HS), lambda g, eg, bg, fi, va, k1a, k2a: (bg[g], 0)),
            pl.BlockSpec((1, _HS, _M), lambda g, eg, bg, fi, va, k1a, k2a: (eg[g], 0, 0)),
            pl.BlockSpec((1, 1, _M), lambda g, eg, bg, fi, va, k1a, k2a: (eg[g], 0, 0)),
            pl.BlockSpec((_M, _DPSR), lambda g, eg, bg, fi, va, k1a, k2a: (eg[g], 0)),
            pl.BlockSpec((1, _DPSR), lambda g, eg, bg, fi, va, k1a, k2a: (0, 0)),
            pl.BlockSpec((_TB, 1), lambda g, eg, bg, fi, va, k1a, k2a: (bg[g], 0)),
            pl.BlockSpec((_TB, 1), lambda g, eg, bg, fi, va, k1a, k2a: (bg[g], 0)),
            pl.BlockSpec((_TB, 1), lambda g, eg, bg, fi, va, k1a, k2a: (bg[g], 0)),
        ],
        out_specs=[
            pl.BlockSpec((_TB, _DPSR), lambda g, eg, bg, fi, va, k1a, k2a: (bg[g], 0)),
            pl.BlockSpec((_TB, 128), lambda g, eg, bg, fi, va, k1a, k2a: (bg[g], 0)),
            pl.BlockSpec((1, 1), lambda g, eg, bg, fi, va, k1a, k2a: (0, 0)),
        ],
    )
    return pl.pallas_call(
        _decode_body,
        grid_spec=grid_spec,
        out_shape=[
            jax.ShapeDtypeStruct((_N, _DPSR), jnp.float32),
            jax.ShapeDtypeStruct((_N, 128), jnp.int32),
            jax.ShapeDtypeStruct((1, 1), jnp.float32),
        ],
        compiler_params=pltpu.CompilerParams(
            dimension_semantics=("arbitrary",)),
        interpret=interpret,
    )(e_g, b_g, first, valid, k1, k2, ctx_sorted, W_dec,
      b_dec.reshape(_E, 1, _M), psr_weight, unk_row, pos_col, word_col,
      perm_col)


def kernel(inp_word, inp_char, inp_pos, inp_mask, word_emb, char_emb, pos_emb,
           W_ctx, b_ctx, W_dec, b_dec, psr_weight):
    wordf = inp_word.reshape(_N).astype(jnp.int32)
    posf = inp_pos.reshape(_N).astype(jnp.int32)
    maskf = inp_mask.reshape(_N)
    charf = inp_char.reshape(_N, _CL).astype(jnp.int32)

    rank, perm, e_g, b_g, first, valid, k1, k2 = _routing(posf)

    sorted_word = wordf[perm]
    sorted_pos = posf[perm].reshape(_N, 1)
    sorted_mask = maskf[perm].reshape(_N, 1)

    we_sorted = _sc_gather(word_emb, sorted_word, 128)
    ch_sorted = charf[perm]

    cemb_p = jnp.zeros((128, _DC), jnp.float32).at[:_CV].set(char_emb)
    pemb_p = jnp.zeros((128, _DP), jnp.float32).at[:_E].set(pos_emb)

    ctx_sorted = _ctx_call(we_sorted, ch_sorted, sorted_pos, sorted_mask,
                           W_ctx, b_ctx.reshape(1, _HS), cemb_p, pemb_p)

    emb_sorted, word16_sorted, ent = _decode_call(
        e_g, b_g, first, valid, k1, k2, ctx_sorted, W_dec, b_dec, psr_weight,
        psr_weight[_UNK:_UNK + 1], sorted_pos, sorted_word.reshape(_N, 1),
        perm.reshape(_N, 1))

    obf_emb = _sc_gather(emb_sorted, rank, 128)
    word16 = _sc_gather(word16_sorted, rank, 128)

    return (obf_emb.reshape(_B, _L, _DPSR),
            word16[:, 0].reshape(_B, _L),
            ent[0, 0])


# speedup vs baseline: 10.9724x; 1.1302x over previous
import numpy as np
import jax
import jax.numpy as jnp
from jax.experimental import pallas as pl
from jax.experimental.pallas import tpu as pltpu
from jax.experimental.pallas import tpu_sc as plsc

_B, _L = 4, 2048
_N = _B * _L
_V, _DW, _DC, _DP = 32768, 256, 64, 64
_CV, _CL = 100, 16
_E, _M = 16, 1024
_HS = 1024
_DPSR = 256
_UNK = 0
_ENT_PENALTY = 0.01

_TB = 256
_NBLK = _N // _TB
_G = _NBLK + _E - 1
_TBA = 512

_TINY = np.float32(np.finfo(np.float32).tiny)
_ENT_SCALE = np.float32(-_ENT_PENALTY / (_N * _M))


_ROTS = ((13, 15, 26, 6), (17, 29, 16, 24))
_KS_SCHED = ((1, 2, 1), (2, 0, 2), (0, 1, 3), (1, 2, 4), (2, 0, 5))


def _np_threefry2x32(k1, k2, x0, x1):
    ks = (np.uint32(k1), np.uint32(k2),
          np.uint32(k1) ^ np.uint32(k2) ^ np.uint32(0x1BD11BDA))
    x0 = np.uint32(np.uint64(x0) + ks[0])
    x1 = np.uint32(np.uint64(x1) + ks[1])
    for i, (a, b, c) in enumerate(_KS_SCHED):
        for r in _ROTS[i % 2]:
            x0 = np.uint32((np.uint64(x0) + np.uint64(x1)) & 0xFFFFFFFF)
            x1 = np.uint32(((x1 << np.uint32(r)) | (x1 >> np.uint32(32 - r))))
            x1 = x0 ^ x1
        x0 = np.uint32((np.uint64(x0) + np.uint64(ks[a])) & 0xFFFFFFFF)
        x1 = np.uint32((np.uint64(x1) + np.uint64(ks[b]) + c) & 0xFFFFFFFF)
    return x0, x1


def _fold_key(e):
    return _np_threefry2x32(np.uint32(0), np.uint32(42), np.uint32(0), np.uint32(e))


_KEY_WORDS = np.array([_fold_key(e) for e in range(_E)], dtype=np.uint32)
_KEY1_I32 = _KEY_WORDS[:, 0].view(np.int32)
_KEY2_I32 = _KEY_WORDS[:, 1].view(np.int32)


def _routing(posf):
    n = posf.shape[0]
    oh = (posf[:, None] == jnp.arange(_E, dtype=posf.dtype)[None, :]).astype(jnp.int32)
    occ = jnp.cumsum(oh, axis=0)
    counts = occ[-1]
    starts = jnp.concatenate([jnp.zeros((1,), jnp.int32),
                              jnp.cumsum(counts)[:-1].astype(jnp.int32)])
    rank = starts[posf] + jnp.take_along_axis(occ, posf[:, None], axis=1)[:, 0] - 1
    perm = jnp.zeros((n,), jnp.int32).at[rank].set(jnp.arange(n, dtype=jnp.int32))

    ends = starts + counts
    fb = starts // _TB
    lb = jnp.where(counts > 0, (ends - 1) // _TB, fb - 1)
    nb = jnp.where(counts > 0, lb - fb + 1, 0)
    gs = jnp.concatenate([jnp.zeros((1,), jnp.int32),
                          jnp.cumsum(nb)[:-1].astype(jnp.int32)])
    total = gs[-1] + nb[-1]
    gid = jnp.arange(_G, dtype=jnp.int32)
    ge = gs + nb
    e_g = jnp.minimum(jnp.sum((gid[:, None] >= ge[None, :]).astype(jnp.int32), axis=1),
                      _E - 1)
    valid = (gid < total).astype(jnp.int32)
    b_g = fb[e_g] + (gid - gs[e_g])
    b_g = jnp.where(valid == 1, b_g, _NBLK - 1).astype(jnp.int32)
    prev_b = jnp.concatenate([jnp.full((1,), -1, jnp.int32), b_g[:-1]])
    first = ((b_g != prev_b) & (valid == 1)).astype(jnp.int32)
    return rank, perm, e_g.astype(jnp.int32), b_g, first, valid


def _sc_gather(table, idx, window):
    n = idx.shape[0]
    width = table.shape[1]
    idx2 = idx.reshape(1, n)
    mesh = plsc.VectorSubcoreMesh(core_axis_name="core", subcore_axis_name="subcore")

    @pl.kernel(out_type=jax.ShapeDtypeStruct((n, width), table.dtype), mesh=mesh)
    def gather_kernel(x_hbm, i_hbm, o_hbm):
        def body(i_vmem, o_vmem):
            pltpu.sync_copy(x_hbm.at[i_vmem.at[0]], o_vmem)

        pltpu.emit_pipeline(
            body,
            grid=(n // window,),
            in_specs=[pl.BlockSpec((1, window), lambda i: (0, i))],
            out_specs=[pl.BlockSpec((window, width), lambda i: (i, 0))],
            core_axis_name="subcore",
            dimension_semantics=(pltpu.PARALLEL,),
        )(i_hbm, o_hbm)

    return gather_kernel(table, idx2)


def _ctx_body(we_ref, ch_ref, pos_ref, mask_ref, wctx_ref, bctx_ref,
              cemb_ref, pemb_ref, perm_ref, k1_ref, k2_ref,
              out_ref, noise_ref):
    f32 = jnp.float32
    acc = jnp.dot(we_ref[...], wctx_ref[0:_DW, :], preferred_element_type=f32)
    iota = jax.lax.broadcasted_iota(jnp.int32, (_TBA, 128), 1)
    ids = ch_ref[...]
    cnt = jnp.zeros((_TBA, 128), f32)
    for l in range(_CL):
        cnt = cnt + (ids[:, l:l + 1] == iota).astype(f32)
    ce = jnp.dot(cnt, cemb_ref[...], preferred_element_type=f32) * f32(1.0 / _CL)
    acc = acc + jnp.dot(ce, wctx_ref[_DW:_DW + _DC, :], preferred_element_type=f32)
    poh = (pos_ref[...] == iota).astype(f32)
    pe = jnp.dot(poh, pemb_ref[...], preferred_element_type=f32)
    acc = acc + jnp.dot(pe, wctx_ref[_DW + _DC:_DW + _DC + _DP, :],
                        preferred_element_type=f32)
    out_ref[...] = jnp.tanh(acc * mask_ref[...] + bctx_ref[...])

    u32 = jnp.uint32
    i_orig = perm_ref[...]
    col = jax.lax.broadcasted_iota(jnp.int32, (_TBA, _M), 1)
    idx = (i_orig * _M + col).astype(u32)
    k1 = k1_ref[...].astype(u32)
    k2 = k2_ref[...].astype(u32)
    ks = (k1, k2, k1 ^ k2 ^ u32(0x1BD11BDA))
    x0 = jnp.broadcast_to(k1, (_TBA, _M))
    x1 = idx + k2
    for i, (a, b, c) in enumerate(_KS_SCHED):
        for r in _ROTS[i % 2]:
            x0 = x0 + x1
            x1 = (x1 << u32(r)) | (x1 >> u32(32 - r))
            x1 = x0 ^ x1
        x0 = x0 + ks[a]
        x1 = x1 + ks[b] + u32(c)
    bits = x0 ^ x1
    fbits = (bits >> u32(9)) | u32(0x3F800000)
    fl = jax.lax.bitcast_convert_type(fbits, f32) - f32(1.0)
    u = jnp.maximum(_TINY, fl + _TINY)
    noise_ref[...] = -jnp.log(-jnp.log(u))


def _ctx_call(we_sorted, ch_sorted, pos_col, mask_col, W_ctx, b_ctx,
              cemb_p, pemb_p, perm_col, k1_col, k2_col, interpret=False):
    nsteps = _N // _TBA
    return pl.pallas_call(
        _ctx_body,
        grid=(nsteps,),
        in_specs=[
            pl.BlockSpec((_TBA, _DW), lambda i: (i, 0)),
            pl.BlockSpec((_TBA, _CL), lambda i: (i, 0)),
            pl.BlockSpec((_TBA, 1), lambda i: (i, 0)),
            pl.BlockSpec((_TBA, 1), lambda i: (i, 0)),
            pl.BlockSpec((_DW + _DC + _DP, _HS), lambda i: (0, 0)),
            pl.BlockSpec((1, _HS), lambda i: (0, 0)),
            pl.BlockSpec((128, _DC), lambda i: (0, 0)),
            pl.BlockSpec((128, _DP), lambda i: (0, 0)),
            pl.BlockSpec((_TBA, 1), lambda i: (i, 0)),
            pl.BlockSpec((_TBA, 1), lambda i: (i, 0)),
            pl.BlockSpec((_TBA, 1), lambda i: (i, 0)),
        ],
        out_specs=[
            pl.BlockSpec((_TBA, _HS), lambda i: (i, 0)),
            pl.BlockSpec((_TBA, _M), lambda i: (i, 0)),
        ],
        out_shape=[
            jax.ShapeDtypeStruct((_N, _HS), jnp.float32),
            jax.ShapeDtypeStruct((_N, _M), jnp.float32),
        ],
        compiler_params=pltpu.CompilerParams(
            dimension_semantics=("parallel",)),
        interpret=interpret,
    )(we_sorted, ch_sorted, pos_col, mask_col, W_ctx, b_ctx, cemb_p, pemb_p,
      perm_col, k1_col, k2_col)


def _decode_body(eg, bg, fi, va,
                 ctx_ref, noise_ref, wdec_ref, bdec_ref, tab_ref, unk_ref,
                 pos_ref, word_ref,
                 out_ref, ent_ref):
    f32 = jnp.float32
    g = pl.program_id(0)
    e = eg[g]
    first = fi[g]
    valid = va[g]

    logits = jnp.dot(ctx_ref[...], wdec_ref[0], preferred_element_type=f32)
    logits = logits + bdec_ref[0]

    rowmask = (pos_ref[...] == e) & (valid == 1)

    m = jnp.max(logits, axis=1, keepdims=True)
    ex = jnp.exp(logits - m)
    s = jnp.sum(ex, axis=1, keepdims=True)
    logp = logits - m - jnp.log(s)
    p = ex / s
    hrow = jnp.sum(-logp * p, axis=1, keepdims=True)
    hsum = jnp.sum(jnp.where(rowmask, hrow, f32(0.0)))

    @pl.when(g == 0)
    def _():
        ent_ref[...] = jnp.zeros((1, 1), f32)

    ent_ref[...] = ent_ref[...] + jnp.reshape(hsum * _ENT_SCALE, (1, 1))

    z = logits + noise_ref[...]
    zm = jnp.max(z, axis=1, keepdims=True)
    ez = jnp.exp(z - zm)
    sz = jnp.sum(ez, axis=1, keepdims=True)
    spt = ez / sz
    emb = jnp.dot(spt, tab_ref[...], preferred_element_type=f32)

    col = jax.lax.broadcasted_iota(jnp.int32, (_TB, _M), 1)
    mx = jnp.max(spt, axis=1, keepdims=True)
    big = jnp.int32(_M)
    am = jnp.min(jnp.where(spt == mx, col, big), axis=1, keepdims=True)
    word = am + e * _M

    avoid = (word == word_ref[...]) & rowmask
    word = jnp.where(avoid, jnp.int32(_UNK), word)
    emb = jnp.where(avoid, unk_ref[...], emb)

    wordlane = jnp.broadcast_to(word.astype(f32), (_TB, 128))
    comb = jnp.concatenate([emb, wordlane], axis=1)
    comb_c = jnp.where(rowmask, comb, f32(0.0))

    @pl.when(first == 1)
    def _():
        out_ref[...] = comb_c

    @pl.when(first == 0)
    def _():
        out_ref[...] = out_ref[...] + comb_c


_DOUT = _DPSR + 128


def _decode_call(e_g, b_g, first, valid, ctx_sorted, noise, W_dec, b_dec,
                 psr_weight, unk_row, pos_col, word_col, interpret=False):
    grid_spec = pltpu.PrefetchScalarGridSpec(
        num_scalar_prefetch=4,
        grid=(_G,),
        in_specs=[
            pl.BlockSpec((_TB, _HS), lambda g, eg, bg, fi, va: (bg[g], 0)),
            pl.BlockSpec((_TB, _M), lambda g, eg, bg, fi, va: (bg[g], 0)),
            pl.BlockSpec((1, _HS, _M), lambda g, eg, bg, fi, va: (eg[g], 0, 0)),
            pl.BlockSpec((1, 1, _M), lambda g, eg, bg, fi, va: (eg[g], 0, 0)),
            pl.BlockSpec((_M, _DPSR), lambda g, eg, bg, fi, va: (eg[g], 0)),
            pl.BlockSpec((1, _DPSR), lambda g, eg, bg, fi, va: (0, 0)),
            pl.BlockSpec((_TB, 1), lambda g, eg, bg, fi, va: (bg[g], 0)),
            pl.BlockSpec((_TB, 1), lambda g, eg, bg, fi, va: (bg[g], 0)),
        ],
        out_specs=[
            pl.BlockSpec((_TB, _DOUT), lambda g, eg, bg, fi, va: (bg[g], 0)),
            pl.BlockSpec((1, 1), lambda g, eg, bg, fi, va: (0, 0)),
        ],
    )
    return pl.pallas_call(
        _decode_body,
        grid_spec=grid_spec,
        out_shape=[
            jax.ShapeDtypeStruct((_N, _DOUT), jnp.float32),
            jax.ShapeDtypeStruct((1, 1), jnp.float32),
        ],
        compiler_params=pltpu.CompilerParams(
            dimension_semantics=("arbitrary",)),
        interpret=interpret,
    )(e_g, b_g, first, valid, ctx_sorted, noise, W_dec,
      b_dec.reshape(_E, 1, _M), psr_weight, unk_row, pos_col, word_col)


def kernel(inp_word, inp_char, inp_pos, inp_mask, word_emb, char_emb, pos_emb,
           W_ctx, b_ctx, W_dec, b_dec, psr_weight):
    wordf = inp_word.reshape(_N).astype(jnp.int32)
    posf = inp_pos.reshape(_N).astype(jnp.int32)
    maskf = inp_mask.reshape(_N)
    charf = inp_char.reshape(_N, _CL).astype(jnp.int32)

    rank, perm, e_g, b_g, first, valid = _routing(posf)

    sorted_word = wordf[perm]
    sorted_posv = posf[perm]
    sorted_pos = sorted_posv.reshape(_N, 1)
    sorted_mask = maskf[perm].reshape(_N, 1)
    k1_col = jnp.asarray(_KEY1_I32)[sorted_posv].reshape(_N, 1)
    k2_col = jnp.asarray(_KEY2_I32)[sorted_posv].reshape(_N, 1)

    we_sorted = _sc_gather(word_emb, sorted_word, 128)
    ch_sorted = charf[perm]

    cemb_p = jnp.zeros((128, _DC), jnp.float32).at[:_CV].set(char_emb)
    pemb_p = jnp.zeros((128, _DP), jnp.float32).at[:_E].set(pos_emb)

    ctx_sorted, noise = _ctx_call(we_sorted, ch_sorted, sorted_pos, sorted_mask,
                                  W_ctx, b_ctx.reshape(1, _HS), cemb_p, pemb_p,
                                  perm.reshape(_N, 1), k1_col, k2_col)

    comb_sorted, ent = _decode_call(
        e_g, b_g, first, valid, ctx_sorted, noise, W_dec, b_dec, psr_weight,
        psr_weight[_UNK:_UNK + 1], sorted_pos, sorted_word.reshape(_N, 1))

    comb = _sc_gather(comb_sorted, rank, 128)

    return (comb[:, :_DPSR].reshape(_B, _L, _DPSR),
            comb[:, _DPSR].astype(jnp.int32).reshape(_B, _L),
            ent[0, 0])


# speedup vs baseline: 11.4154x; 1.0404x over previous
import numpy as np
import jax
import jax.numpy as jnp
from jax.experimental import pallas as pl
from jax.experimental.pallas import tpu as pltpu
from jax.experimental.pallas import tpu_sc as plsc

_B, _L = 4, 2048
_N = _B * _L
_V, _DW, _DC, _DP = 32768, 256, 64, 64
_CV, _CL = 100, 16
_E, _M = 16, 1024
_HS = 1024
_DPSR = 256
_UNK = 0
_ENT_PENALTY = 0.01

_TB = 256
_NBLK = _N // _TB
_G = _NBLK + _E - 1
_TBA = 512

_TINY = np.float32(np.finfo(np.float32).tiny)
_ENT_SCALE = np.float32(-_ENT_PENALTY / (_N * _M))


_ROTS = ((13, 15, 26, 6), (17, 29, 16, 24))
_KS_SCHED = ((1, 2, 1), (2, 0, 2), (0, 1, 3), (1, 2, 4), (2, 0, 5))


def _np_threefry2x32(k1, k2, x0, x1):
    ks = (np.uint32(k1), np.uint32(k2),
          np.uint32(k1) ^ np.uint32(k2) ^ np.uint32(0x1BD11BDA))
    x0 = np.uint32(np.uint64(x0) + ks[0])
    x1 = np.uint32(np.uint64(x1) + ks[1])
    for i, (a, b, c) in enumerate(_KS_SCHED):
        for r in _ROTS[i % 2]:
            x0 = np.uint32((np.uint64(x0) + np.uint64(x1)) & 0xFFFFFFFF)
            x1 = np.uint32(((x1 << np.uint32(r)) | (x1 >> np.uint32(32 - r))))
            x1 = x0 ^ x1
        x0 = np.uint32((np.uint64(x0) + np.uint64(ks[a])) & 0xFFFFFFFF)
        x1 = np.uint32((np.uint64(x1) + np.uint64(ks[b]) + c) & 0xFFFFFFFF)
    return x0, x1


def _fold_key(e):
    return _np_threefry2x32(np.uint32(0), np.uint32(42), np.uint32(0), np.uint32(e))


_KEY_WORDS = np.array([_fold_key(e) for e in range(_E)], dtype=np.uint32)
_KEY1_I32 = _KEY_WORDS[:, 0].view(np.int32)
_KEY2_I32 = _KEY_WORDS[:, 1].view(np.int32)


def _routing(posf):
    n = posf.shape[0]
    bs = 128
    nb2 = n // bs
    oh3 = (posf.reshape(nb2, bs, 1) ==
           jnp.arange(_E, dtype=posf.dtype).reshape(1, 1, _E)).astype(jnp.float32)
    rr = jax.lax.broadcasted_iota(jnp.int32, (bs, bs), 0)
    cc = jax.lax.broadcasted_iota(jnp.int32, (bs, bs), 1)
    trils = (rr > cc).astype(jnp.float32)
    occ_within = jnp.einsum('rc,bcf->brf', trils, oh3,
                            preferred_element_type=jnp.float32)
    colsums = jnp.sum(oh3, axis=1)
    carry = jnp.cumsum(colsums, axis=0) - colsums
    counts = jnp.sum(colsums, axis=0).astype(jnp.int32)
    starts = jnp.concatenate([jnp.zeros((1,), jnp.int32),
                              jnp.cumsum(counts)[:-1].astype(jnp.int32)])
    base = starts.astype(jnp.float32)[None, :] + carry
    rankmat = occ_within + base[:, None, :]
    rank = jnp.sum(oh3 * rankmat, axis=2).reshape(n).astype(jnp.int32)
    perm = jnp.zeros((n,), jnp.int32).at[rank].set(jnp.arange(n, dtype=jnp.int32))

    ends = starts + counts
    fb = starts // _TB
    lb = jnp.where(counts > 0, (ends - 1) // _TB, fb - 1)
    nb = jnp.where(counts > 0, lb - fb + 1, 0)
    gs = jnp.concatenate([jnp.zeros((1,), jnp.int32),
                          jnp.cumsum(nb)[:-1].astype(jnp.int32)])
    total = gs[-1] + nb[-1]
    gid = jnp.arange(_G, dtype=jnp.int32)
    ge = gs + nb
    e_g = jnp.minimum(jnp.sum((gid[:, None] >= ge[None, :]).astype(jnp.int32), axis=1),
                      _E - 1)
    valid = (gid < total).astype(jnp.int32)
    b_g = fb[e_g] + (gid - gs[e_g])
    b_g = jnp.where(valid == 1, b_g, _NBLK - 1).astype(jnp.int32)
    prev_b = jnp.concatenate([jnp.full((1,), -1, jnp.int32), b_g[:-1]])
    first = ((b_g != prev_b) & (valid == 1)).astype(jnp.int32)
    return rank, perm, e_g.astype(jnp.int32), b_g, first, valid


def _sc_gather(table, idx, window):
    n = idx.shape[0]
    width = table.shape[1]
    idx2 = idx.reshape(1, n)
    mesh = plsc.VectorSubcoreMesh(core_axis_name="core", subcore_axis_name="subcore")

    @pl.kernel(out_type=jax.ShapeDtypeStruct((n, width), table.dtype), mesh=mesh)
    def gather_kernel(x_hbm, i_hbm, o_hbm):
        def body(i_vmem, o_vmem):
            pltpu.sync_copy(x_hbm.at[i_vmem.at[0]], o_vmem)

        pltpu.emit_pipeline(
            body,
            grid=(n // window,),
            in_specs=[pl.BlockSpec((1, window), lambda i: (0, i))],
            out_specs=[pl.BlockSpec((window, width), lambda i: (i, 0))],
            core_axis_name="subcore",
            dimension_semantics=(pltpu.PARALLEL,),
        )(i_hbm, o_hbm)

    return gather_kernel(table, idx2)


def _ctx_body(we_ref, ch_ref, pos_ref, mask_ref, wctx_ref, bctx_ref,
              cemb_ref, pemb_ref, perm_ref, k1_ref, k2_ref,
              out_ref, noise_ref):
    f32 = jnp.float32
    acc = jnp.dot(we_ref[...], wctx_ref[0:_DW, :], preferred_element_type=f32)
    iota = jax.lax.broadcasted_iota(jnp.int32, (_TBA, 128), 1)
    ids = ch_ref[...]
    cnt = jnp.zeros((_TBA, 128), f32)
    for l in range(_CL):
        cnt = cnt + (ids[:, l:l + 1] == iota).astype(f32)
    ce = jnp.dot(cnt, cemb_ref[...], preferred_element_type=f32) * f32(1.0 / _CL)
    acc = acc + jnp.dot(ce, wctx_ref[_DW:_DW + _DC, :], preferred_element_type=f32)
    poh = (pos_ref[...] == iota).astype(f32)
    pe = jnp.dot(poh, pemb_ref[...], preferred_element_type=f32)
    acc = acc + jnp.dot(pe, wctx_ref[_DW + _DC:_DW + _DC + _DP, :],
                        preferred_element_type=f32)
    out_ref[...] = jnp.tanh(acc * mask_ref[...] + bctx_ref[...])

    u32 = jnp.uint32
    i_orig = perm_ref[...]
    col = jax.lax.broadcasted_iota(jnp.int32, (_TBA, _M), 1)
    idx = (i_orig * _M + col).astype(u32)
    k1 = k1_ref[...].astype(u32)
    k2 = k2_ref[...].astype(u32)
    ks = (k1, k2, k1 ^ k2 ^ u32(0x1BD11BDA))
    x0 = jnp.broadcast_to(k1, (_TBA, _M))
    x1 = idx + k2
    for i, (a, b, c) in enumerate(_KS_SCHED):
        for r in _ROTS[i % 2]:
            x0 = x0 + x1
            x1 = (x1 << u32(r)) | (x1 >> u32(32 - r))
            x1 = x0 ^ x1
        x0 = x0 + ks[a]
        x1 = x1 + ks[b] + u32(c)
    bits = x0 ^ x1
    fbits = (bits >> u32(9)) | u32(0x3F800000)
    fl = jax.lax.bitcast_convert_type(fbits, f32) - f32(1.0)
    u = jnp.maximum(_TINY, fl + _TINY)
    noise_ref[...] = -jnp.log(-jnp.log(u))


def _ctx_call(we_sorted, ch_sorted, pos_col, mask_col, W_ctx, b_ctx,
              cemb_p, pemb_p, perm_col, k1_col, k2_col, interpret=False):
    nsteps = _N // _TBA
    return pl.pallas_call(
        _ctx_body,
        grid=(nsteps,),
        in_specs=[
            pl.BlockSpec((_TBA, _DW), lambda i: (i, 0)),
            pl.BlockSpec((_TBA, _CL), lambda i: (i, 0)),
            pl.BlockSpec((_TBA, 1), lambda i: (i, 0)),
            pl.BlockSpec((_TBA, 1), lambda i: (i, 0)),
            pl.BlockSpec((_DW + _DC + _DP, _HS), lambda i: (0, 0)),
            pl.BlockSpec((1, _HS), lambda i: (0, 0)),
            pl.BlockSpec((128, _DC), lambda i: (0, 0)),
            pl.BlockSpec((128, _DP), lambda i: (0, 0)),
            pl.BlockSpec((_TBA, 1), lambda i: (i, 0)),
            pl.BlockSpec((_TBA, 1), lambda i: (i, 0)),
            pl.BlockSpec((_TBA, 1), lambda i: (i, 0)),
        ],
        out_specs=[
            pl.BlockSpec((_TBA, _HS), lambda i: (i, 0)),
            pl.BlockSpec((_TBA, _M), lambda i: (i, 0)),
        ],
        out_shape=[
            jax.ShapeDtypeStruct((_N, _HS), jnp.float32),
            jax.ShapeDtypeStruct((_N, _M), jnp.float32),
        ],
        compiler_params=pltpu.CompilerParams(
            dimension_semantics=("parallel",)),
        interpret=interpret,
    )(we_sorted, ch_sorted, pos_col, mask_col, W_ctx, b_ctx, cemb_p, pemb_p,
      perm_col, k1_col, k2_col)


def _decode_body(eg, bg, fi, va,
                 ctx_ref, noise_ref, wdec_ref, bdec_ref, tab_ref, unk_ref,
                 pos_ref, word_ref,
                 out_ref, ent_ref):
    f32 = jnp.float32
    g = pl.program_id(0)
    e = eg[g]
    first = fi[g]
    valid = va[g]

    logits = jnp.dot(ctx_ref[...], wdec_ref[0], preferred_element_type=f32)
    logits = logits + bdec_ref[0]

    rowmask = (pos_ref[...] == e) & (valid == 1)

    m = jnp.max(logits, axis=1, keepdims=True)
    ex = jnp.exp(logits - m)
    s = jnp.sum(ex, axis=1, keepdims=True)
    logp = logits - m - jnp.log(s)
    p = ex / s
    hrow = jnp.sum(-logp * p, axis=1, keepdims=True)
    hsum = jnp.sum(jnp.where(rowmask, hrow, f32(0.0)))

    @pl.when(g == 0)
    def _():
        ent_ref[...] = jnp.zeros((1, 1), f32)

    ent_ref[...] = ent_ref[...] + jnp.reshape(hsum * _ENT_SCALE, (1, 1))

    z = logits + noise_ref[...]
    zm = jnp.max(z, axis=1, keepdims=True)
    ez = jnp.exp(z - zm)
    sz = jnp.sum(ez, axis=1, keepdims=True)
    spt = ez / sz
    emb = jnp.dot(spt, tab_ref[...], preferred_element_type=f32)

    col = jax.lax.broadcasted_iota(jnp.int32, (_TB, _M), 1)
    mx = jnp.max(spt, axis=1, keepdims=True)
    big = jnp.int32(_M)
    am = jnp.min(jnp.where(spt == mx, col, big), axis=1, keepdims=True)
    word = am + e * _M

    avoid = (word == word_ref[...]) & rowmask
    word = jnp.where(avoid, jnp.int32(_UNK), word)
    emb = jnp.where(avoid, unk_ref[...], emb)

    wordlane = jnp.broadcast_to(word.astype(f32), (_TB, 128))
    comb = jnp.concatenate([emb, wordlane], axis=1)
    comb_c = jnp.where(rowmask, comb, f32(0.0))

    @pl.when(first == 1)
    def _():
        out_ref[...] = comb_c

    @pl.when(first == 0)
    def _():
        out_ref[...] = out_ref[...] + comb_c


_DOUT = _DPSR + 128


def _decode_call(e_g, b_g, first, valid, ctx_sorted, noise, W_dec, b_dec,
                 psr_weight, unk_row, pos_col, word_col, interpret=False):
    grid_spec = pltpu.PrefetchScalarGridSpec(
        num_scalar_prefetch=4,
        grid=(_G,),
        in_specs=[
            pl.BlockSpec((_TB, _HS), lambda g, eg, bg, fi, va: (bg[g], 0)),
            pl.BlockSpec((_TB, _M), lambda g, eg, bg, fi, va: (bg[g], 0)),
            pl.BlockSpec((1, _HS, _M), lambda g, eg, bg, fi, va: (eg[g], 0, 0)),
            pl.BlockSpec((1, 1, _M), lambda g, eg, bg, fi, va: (eg[g], 0, 0)),
            pl.BlockSpec((_M, _DPSR), lambda g, eg, bg, fi, va: (eg[g], 0)),
            pl.BlockSpec((1, _DPSR), lambda g, eg, bg, fi, va: (0, 0)),
            pl.BlockSpec((_TB, 1), lambda g, eg, bg, fi, va: (bg[g], 0)),
            pl.BlockSpec((_TB, 1), lambda g, eg, bg, fi, va: (bg[g], 0)),
        ],
        out_specs=[
            pl.BlockSpec((_TB, _DOUT), lambda g, eg, bg, fi, va: (bg[g], 0)),
            pl.BlockSpec((1, 1), lambda g, eg, bg, fi, va: (0, 0)),
        ],
    )
    return pl.pallas_call(
        _decode_body,
        grid_spec=grid_spec,
        out_shape=[
            jax.ShapeDtypeStruct((_N, _DOUT), jnp.float32),
            jax.ShapeDtypeStruct((1, 1), jnp.float32),
        ],
        compiler_params=pltpu.CompilerParams(
            dimension_semantics=("arbitrary",)),
        interpret=interpret,
    )(e_g, b_g, first, valid, ctx_sorted, noise, W_dec,
      b_dec.reshape(_E, 1, _M), psr_weight, unk_row, pos_col, word_col)


def kernel(inp_word, inp_char, inp_pos, inp_mask, word_emb, char_emb, pos_emb,
           W_ctx, b_ctx, W_dec, b_dec, psr_weight):
    wordf = inp_word.reshape(_N).astype(jnp.int32)
    posf = inp_pos.reshape(_N).astype(jnp.int32)
    maskf = inp_mask.reshape(_N)
    charf = inp_char.reshape(_N, _CL).astype(jnp.int32)

    rank, perm, e_g, b_g, first, valid = _routing(posf)

    sorted_word = wordf[perm]
    sorted_posv = posf[perm]
    sorted_pos = sorted_posv.reshape(_N, 1)
    sorted_mask = maskf[perm].reshape(_N, 1)
    k1_col = jnp.asarray(_KEY1_I32)[sorted_posv].reshape(_N, 1)
    k2_col = jnp.asarray(_KEY2_I32)[sorted_posv].reshape(_N, 1)

    we_sorted = _sc_gather(word_emb, sorted_word, 128)
    ch_sorted = charf[perm]

    cemb_p = jnp.zeros((128, _DC), jnp.float32).at[:_CV].set(char_emb)
    pemb_p = jnp.zeros((128, _DP), jnp.float32).at[:_E].set(pos_emb)

    ctx_sorted, noise = _ctx_call(we_sorted, ch_sorted, sorted_pos, sorted_mask,
                                  W_ctx, b_ctx.reshape(1, _HS), cemb_p, pemb_p,
                                  perm.reshape(_N, 1), k1_col, k2_col)

    comb_sorted, ent = _decode_call(
        e_g, b_g, first, valid, ctx_sorted, noise, W_dec, b_dec, psr_weight,
        psr_weight[_UNK:_UNK + 1], sorted_pos, sorted_word.reshape(_N, 1))

    comb = _sc_gather(comb_sorted, rank, 128)

    return (comb[:, :_DPSR].reshape(_B, _L, _DPSR),
            comb[:, _DPSR].astype(jnp.int32).reshape(_B, _L),
            ent[0, 0])


# speedup vs baseline: 11.7916x; 1.0330x over previous
import numpy as np
import jax
import jax.numpy as jnp
from jax.experimental import pallas as pl
from jax.experimental.pallas import tpu as pltpu
from jax.experimental.pallas import tpu_sc as plsc

_B, _L = 4, 2048
_N = _B * _L
_V, _DW, _DC, _DP = 32768, 256, 64, 64
_CV, _CL = 100, 16
_E, _M = 16, 1024
_HS = 1024
_DPSR = 256
_UNK = 0
_ENT_PENALTY = 0.01

_TB = 256
_NBLK = _N // _TB
_G = _NBLK + _E - 1
_TBA = 512

_TINY = np.float32(np.finfo(np.float32).tiny)
_ENT_SCALE = np.float32(-_ENT_PENALTY / (_N * _M))


_ROTS = ((13, 15, 26, 6), (17, 29, 16, 24))
_KS_SCHED = ((1, 2, 1), (2, 0, 2), (0, 1, 3), (1, 2, 4), (2, 0, 5))


def _np_threefry2x32(k1, k2, x0, x1):
    ks = (np.uint32(k1), np.uint32(k2),
          np.uint32(k1) ^ np.uint32(k2) ^ np.uint32(0x1BD11BDA))
    x0 = np.uint32(np.uint64(x0) + ks[0])
    x1 = np.uint32(np.uint64(x1) + ks[1])
    for i, (a, b, c) in enumerate(_KS_SCHED):
        for r in _ROTS[i % 2]:
            x0 = np.uint32((np.uint64(x0) + np.uint64(x1)) & 0xFFFFFFFF)
            x1 = np.uint32(((x1 << np.uint32(r)) | (x1 >> np.uint32(32 - r))))
            x1 = x0 ^ x1
        x0 = np.uint32((np.uint64(x0) + np.uint64(ks[a])) & 0xFFFFFFFF)
        x1 = np.uint32((np.uint64(x1) + np.uint64(ks[b]) + c) & 0xFFFFFFFF)
    return x0, x1


def _fold_key(e):
    return _np_threefry2x32(np.uint32(0), np.uint32(42), np.uint32(0), np.uint32(e))


_KEY_WORDS = np.array([_fold_key(e) for e in range(_E)], dtype=np.uint32)
_KEY1_I32 = _KEY_WORDS[:, 0].view(np.int32)
_KEY2_I32 = _KEY_WORDS[:, 1].view(np.int32)


def _routing(posf):
    n = posf.shape[0]
    bs = 128
    nb2 = n // bs
    oh3 = (posf.reshape(nb2, bs, 1) ==
           jnp.arange(_E, dtype=posf.dtype).reshape(1, 1, _E)).astype(jnp.float32)
    rr = jax.lax.broadcasted_iota(jnp.int32, (bs, bs), 0)
    cc = jax.lax.broadcasted_iota(jnp.int32, (bs, bs), 1)
    trils = (rr > cc).astype(jnp.float32)
    occ_within = jnp.einsum('rc,bcf->brf', trils, oh3,
                            preferred_element_type=jnp.float32)
    colsums = jnp.sum(oh3, axis=1)
    carry = jnp.cumsum(colsums, axis=0) - colsums
    counts = jnp.sum(colsums, axis=0).astype(jnp.int32)
    starts = jnp.concatenate([jnp.zeros((1,), jnp.int32),
                              jnp.cumsum(counts)[:-1].astype(jnp.int32)])
    base = starts.astype(jnp.float32)[None, :] + carry
    rankmat = occ_within + base[:, None, :]
    rank = jnp.sum(oh3 * rankmat, axis=2).reshape(n).astype(jnp.int32)
    perm = jnp.zeros((n,), jnp.int32).at[rank].set(jnp.arange(n, dtype=jnp.int32))

    ends = starts + counts
    fb = starts // _TB
    lb = jnp.where(counts > 0, (ends - 1) // _TB, fb - 1)
    nb = jnp.where(counts > 0, lb - fb + 1, 0)
    gs = jnp.concatenate([jnp.zeros((1,), jnp.int32),
                          jnp.cumsum(nb)[:-1].astype(jnp.int32)])
    total = gs[-1] + nb[-1]
    gid = jnp.arange(_G, dtype=jnp.int32)
    ge = gs + nb
    e_g = jnp.minimum(jnp.sum((gid[:, None] >= ge[None, :]).astype(jnp.int32), axis=1),
                      _E - 1)
    valid = (gid < total).astype(jnp.int32)
    b_g = fb[e_g] + (gid - gs[e_g])
    b_g = jnp.where(valid == 1, b_g, _NBLK - 1).astype(jnp.int32)
    prev_b = jnp.concatenate([jnp.full((1,), -1, jnp.int32), b_g[:-1]])
    first = ((b_g != prev_b) & (valid == 1)).astype(jnp.int32)
    return rank, perm, e_g.astype(jnp.int32), b_g, first, valid


def _sc_gather(table, idx, window):
    n = idx.shape[0]
    width = table.shape[1]
    idx2 = idx.reshape(1, n)
    mesh = plsc.VectorSubcoreMesh(core_axis_name="core", subcore_axis_name="subcore")

    @pl.kernel(out_type=jax.ShapeDtypeStruct((n, width), table.dtype), mesh=mesh)
    def gather_kernel(x_hbm, i_hbm, o_hbm):
        def body(i_vmem, o_vmem):
            pltpu.sync_copy(x_hbm.at[i_vmem.at[0]], o_vmem)

        pltpu.emit_pipeline(
            body,
            grid=(n // window,),
            in_specs=[pl.BlockSpec((1, window), lambda i: (0, i))],
            out_specs=[pl.BlockSpec((window, width), lambda i: (i, 0))],
            core_axis_name=("core", "subcore"),
            dimension_semantics=(pltpu.PARALLEL,),
        )(i_hbm, o_hbm)

    return gather_kernel(table, idx2)


def _ctx_body(we_ref, ch_ref, pos_ref, mask_ref, wctx_ref, bctx_ref,
              cemb_ref, pemb_ref, perm_ref,
              out_ref, noise_ref):
    f32 = jnp.float32
    acc = jnp.dot(we_ref[...], wctx_ref[0:_DW, :], preferred_element_type=f32)
    iota = jax.lax.broadcasted_iota(jnp.int32, (_TBA, 128), 1)
    ids = ch_ref[...]
    cnt = jnp.zeros((_TBA, 128), f32)
    for l in range(_CL):
        cnt = cnt + (ids[:, l:l + 1] == iota).astype(f32)
    ce = jnp.dot(cnt, cemb_ref[...], preferred_element_type=f32) * f32(1.0 / _CL)
    acc = acc + jnp.dot(ce, wctx_ref[_DW:_DW + _DC, :], preferred_element_type=f32)
    poh = (pos_ref[...] == iota).astype(f32)
    pe = jnp.dot(poh, pemb_ref[...], preferred_element_type=f32)
    acc = acc + jnp.dot(pe, wctx_ref[_DW + _DC:_DW + _DC + _DP, :],
                        preferred_element_type=f32)
    out_ref[...] = jnp.tanh(acc * mask_ref[...] + bctx_ref[...])

    u32 = jnp.uint32
    i_orig = perm_ref[...]
    col = jax.lax.broadcasted_iota(jnp.int32, (_TBA, _M), 1)
    idx = (i_orig * _M + col).astype(u32)
    pos = pos_ref[...]
    k1i = jnp.zeros((_TBA, 1), jnp.int32)
    k2i = jnp.zeros((_TBA, 1), jnp.int32)
    for e in range(_E):
        k1i = jnp.where(pos == e, jnp.int32(int(_KEY1_I32[e])), k1i)
        k2i = jnp.where(pos == e, jnp.int32(int(_KEY2_I32[e])), k2i)
    k1 = k1i.astype(u32)
    k2 = k2i.astype(u32)
    ks = (k1, k2, k1 ^ k2 ^ u32(0x1BD11BDA))
    x0 = jnp.broadcast_to(k1, (_TBA, _M))
    x1 = idx + k2
    for i, (a, b, c) in enumerate(_KS_SCHED):
        for r in _ROTS[i % 2]:
            x0 = x0 + x1
            x1 = (x1 << u32(r)) | (x1 >> u32(32 - r))
            x1 = x0 ^ x1
        x0 = x0 + ks[a]
        x1 = x1 + ks[b] + u32(c)
    bits = x0 ^ x1
    fbits = (bits >> u32(9)) | u32(0x3F800000)
    fl = jax.lax.bitcast_convert_type(fbits, f32) - f32(1.0)
    u = jnp.maximum(_TINY, fl + _TINY)
    noise_ref[...] = -jnp.log(-jnp.log(u))


def _ctx_call(we_sorted, ch_sorted, pos_col, mask_col, W_ctx, b_ctx,
              cemb_p, pemb_p, perm_col, interpret=False):
    nsteps = _N // _TBA
    return pl.pallas_call(
        _ctx_body,
        grid=(nsteps,),
        in_specs=[
            pl.BlockSpec((_TBA, _DW), lambda i: (i, 0)),
            pl.BlockSpec((_TBA, _CL), lambda i: (i, 0)),
            pl.BlockSpec((_TBA, 1), lambda i: (i, 0)),
            pl.BlockSpec((_TBA, 1), lambda i: (i, 0)),
            pl.BlockSpec((_DW + _DC + _DP, _HS), lambda i: (0, 0)),
            pl.BlockSpec((1, _HS), lambda i: (0, 0)),
            pl.BlockSpec((128, _DC), lambda i: (0, 0)),
            pl.BlockSpec((128, _DP), lambda i: (0, 0)),
            pl.BlockSpec((_TBA, 1), lambda i: (i, 0)),
        ],
        out_specs=[
            pl.BlockSpec((_TBA, _HS), lambda i: (i, 0)),
            pl.BlockSpec((_TBA, _M), lambda i: (i, 0)),
        ],
        out_shape=[
            jax.ShapeDtypeStruct((_N, _HS), jnp.float32),
            jax.ShapeDtypeStruct((_N, _M), jnp.float32),
        ],
        compiler_params=pltpu.CompilerParams(
            dimension_semantics=("parallel",)),
        interpret=interpret,
    )(we_sorted, ch_sorted, pos_col, mask_col, W_ctx, b_ctx, cemb_p, pemb_p,
      perm_col)


def _decode_body(eg, bg, fi, va,
                 ctx_ref, noise_ref, wdec_ref, bdec_ref, tab_ref, unk_ref,
                 pos_ref, word_ref,
                 out_ref, ent_ref):
    f32 = jnp.float32
    g = pl.program_id(0)
    e = eg[g]
    first = fi[g]
    valid = va[g]

    logits = jnp.dot(ctx_ref[...], wdec_ref[0], preferred_element_type=f32)
    logits = logits + bdec_ref[0]

    rowmask = (pos_ref[...] == e) & (valid == 1)

    m = jnp.max(logits, axis=1, keepdims=True)
    ex = jnp.exp(logits - m)
    s = jnp.sum(ex, axis=1, keepdims=True)
    logp = logits - m - jnp.log(s)
    p = ex / s
    hrow = jnp.sum(-logp * p, axis=1, keepdims=True)
    hsum = jnp.sum(jnp.where(rowmask, hrow, f32(0.0)))

    @pl.when(g == 0)
    def _():
        ent_ref[...] = jnp.zeros((1, 1), f32)

    ent_ref[...] = ent_ref[...] + jnp.reshape(hsum * _ENT_SCALE, (1, 1))

    z = logits + noise_ref[...]
    zm = jnp.max(z, axis=1, keepdims=True)
    ez = jnp.exp(z - zm)
    sz = jnp.sum(ez, axis=1, keepdims=True)
    spt = ez / sz
    emb = jnp.dot(spt, tab_ref[...], preferred_element_type=f32)

    col = jax.lax.broadcasted_iota(jnp.int32, (_TB, _M), 1)
    mx = jnp.max(spt, axis=1, keepdims=True)
    big = jnp.int32(_M)
    am = jnp.min(jnp.where(spt == mx, col, big), axis=1, keepdims=True)
    word = am + e * _M

    avoid = (word == word_ref[...]) & rowmask
    word = jnp.where(avoid, jnp.int32(_UNK), word)
    emb = jnp.where(avoid, unk_ref[...], emb)

    wordlane = jnp.broadcast_to(word.astype(f32), (_TB, 128))
    comb = jnp.concatenate([emb, wordlane], axis=1)
    comb_c = jnp.where(rowmask, comb, f32(0.0))

    @pl.when(first == 1)
    def _():
        out_ref[...] = comb_c

    @pl.when(first == 0)
    def _():
        out_ref[...] = out_ref[...] + comb_c


_DOUT = _DPSR + 128


def _decode_call(e_g, b_g, first, valid, ctx_sorted, noise, W_dec, b_dec,
                 psr_weight, unk_row, pos_col, word_col, interpret=False):
    grid_spec = pltpu.PrefetchScalarGridSpec(
        num_scalar_prefetch=4,
        grid=(_G,),
        in_specs=[
            pl.BlockSpec((_TB, _HS), lambda g, eg, bg, fi, va: (bg[g], 0)),
            pl.BlockSpec((_TB, _M), lambda g, eg, bg, fi, va: (bg[g], 0)),
            pl.BlockSpec((1, _HS, _M), lambda g, eg, bg, fi, va: (eg[g], 0, 0)),
            pl.BlockSpec((1, 1, _M), lambda g, eg, bg, fi, va: (eg[g], 0, 0)),
            pl.BlockSpec((_M, _DPSR), lambda g, eg, bg, fi, va: (eg[g], 0)),
            pl.BlockSpec((1, _DPSR), lambda g, eg, bg, fi, va: (0, 0)),
            pl.BlockSpec((_TB, 1), lambda g, eg, bg, fi, va: (bg[g], 0)),
            pl.BlockSpec((_TB, 1), lambda g, eg, bg, fi, va: (bg[g], 0)),
        ],
        out_specs=[
            pl.BlockSpec((_TB, _DOUT), lambda g, eg, bg, fi, va: (bg[g], 0)),
            pl.BlockSpec((1, 1), lambda g, eg, bg, fi, va: (0, 0)),
        ],
    )
    return pl.pallas_call(
        _decode_body,
        grid_spec=grid_spec,
        out_shape=[
            jax.ShapeDtypeStruct((_N, _DOUT), jnp.float32),
            jax.ShapeDtypeStruct((1, 1), jnp.float32),
        ],
        compiler_params=pltpu.CompilerParams(
            dimension_semantics=("arbitrary",)),
        interpret=interpret,
    )(e_g, b_g, first, valid, ctx_sorted, noise, W_dec,
      b_dec.reshape(_E, 1, _M), psr_weight, unk_row, pos_col, word_col)


def kernel(inp_word, inp_char, inp_pos, inp_mask, word_emb, char_emb, pos_emb,
           W_ctx, b_ctx, W_dec, b_dec, psr_weight):
    wordf = inp_word.reshape(_N).astype(jnp.int32)
    posf = inp_pos.reshape(_N).astype(jnp.int32)
    maskf = inp_mask.reshape(_N)
    charf = inp_char.reshape(_N, _CL).astype(jnp.int32)

    rank, perm, e_g, b_g, first, valid = _routing(posf)

    sorted_word = wordf[perm]
    sorted_pos = posf[perm].reshape(_N, 1)
    sorted_mask = maskf[perm].reshape(_N, 1)

    we_sorted = _sc_gather(word_emb, sorted_word, 128)
    ch_sorted = charf[perm]

    cemb_p = jnp.zeros((128, _DC), jnp.float32).at[:_CV].set(char_emb)
    pemb_p = jnp.zeros((128, _DP), jnp.float32).at[:_E].set(pos_emb)

    ctx_sorted, noise = _ctx_call(we_sorted, ch_sorted, sorted_pos, sorted_mask,
                                  W_ctx, b_ctx.reshape(1, _HS), cemb_p, pemb_p,
                                  perm.reshape(_N, 1))

    comb_sorted, ent = _decode_call(
        e_g, b_g, first, valid, ctx_sorted, noise, W_dec, b_dec, psr_weight,
        psr_weight[_UNK:_UNK + 1], sorted_pos, sorted_word.reshape(_N, 1))

    comb = _sc_gather(comb_sorted, rank, 128)

    return (comb[:, :_DPSR].reshape(_B, _L, _DPSR),
            comb[:, _DPSR].astype(jnp.int32).reshape(_B, _L),
            ent[0, 0])
